# Initial kernel scaffold; baseline (speedup 1.0000x reference)
import jax
import jax.numpy as jnp
from jax.experimental import pallas as pl


def kernel(x, Ws, bs):
    raise NotImplementedError("write your pallas kernel here")


# speedup vs baseline: 1.1473x; 1.1473x over previous
import functools

import jax
import jax.numpy as jnp
from jax.experimental import pallas as pl
from jax.experimental.pallas import tpu as pltpu

_N_LAYERS = 20
_D = 768
_BT = 512


def _mlp_body(x_ref, ws_ref, bs_ref, o_ref):
    h = x_ref[...]
    for i in range(_N_LAYERS):
        w = ws_ref[i, :, :]
        b = bs_ref[i, :]
        h = jax.lax.dot_general(
            h, w, (((1,), (1,)), ((), ())),
            preferred_element_type=jnp.float32,
        ) + b[None, :]
    o_ref[...] = h


@jax.jit
def kernel(x, Ws, bs):
    n_tok, d = x.shape
    grid = (n_tok // _BT,)
    return pl.pallas_call(
        _mlp_body,
        grid=grid,
        in_specs=[
            pl.BlockSpec((_BT, d), lambda t: (t, 0)),
            pl.BlockSpec((_N_LAYERS, d, d), lambda t: (0, 0, 0)),
            pl.BlockSpec((_N_LAYERS, d), lambda t: (0, 0)),
        ],
        out_specs=pl.BlockSpec((_BT, d), lambda t: (t, 0)),
        out_shape=jax.ShapeDtypeStruct((n_tok, d), jnp.float32),
        compiler_params=pltpu.CompilerParams(
            dimension_semantics=("parallel",),
        ),
    )(x, Ws, bs)


# speedup vs baseline: 4.0110x; 3.4959x over previous
import jax
import jax.numpy as jnp
from jax.experimental import pallas as pl
from jax.experimental.pallas import tpu as pltpu

_N_LAYERS = 20
_D = 768
_BT = 4096


def _dot(a, b):
    return jax.lax.dot_general(
        a, b, (((1,), (0,)), ((), ())), preferred_element_type=jnp.float32)


def _combine_body(whi_ref, wlo_ref, b_ref, ow_ref, ob_ref, m_ref, r_ref):
    i = pl.program_id(0)
    whi = whi_ref[0, :, :]
    wlo = wlo_ref[0, :, :]
    b = b_ref[0, :, :]

    @pl.when(i == 0)
    def _init():
        m_ref[...] = whi.astype(jnp.float32) + wlo.astype(jnp.float32)
        r_ref[...] = b

    @pl.when(i > 0)
    def _step():
        m = m_ref[...]
        mhi = m.astype(jnp.bfloat16)
        mlo = (m - mhi.astype(jnp.float32)).astype(jnp.bfloat16)
        m_ref[...] = _dot(mhi, whi) + (_dot(mhi, wlo) + _dot(mlo, whi))
        rb = r_ref[...].astype(jnp.bfloat16)
        r_ref[...] = _dot(rb, whi) + _dot(rb, wlo) + b

    @pl.when(i == _N_LAYERS - 1)
    def _emit():
        ow_ref[...] = m_ref[...]
        ob_ref[...] = r_ref[...]


def _apply_body(x_ref, wct_ref, bc_ref, o_ref):
    o_ref[...] = _dot(x_ref[...], wct_ref[...]) + bc_ref[0, :][None, :]


@jax.jit
def kernel(x, Ws, bs):
    n_tok, d = x.shape
    wt = Ws.transpose(0, 2, 1)
    w_hi = wt.astype(jnp.bfloat16)
    w_lo = (wt - w_hi.astype(jnp.float32)).astype(jnp.bfloat16)
    bs3 = bs.reshape(_N_LAYERS, 1, d)

    wct, bc = pl.pallas_call(
        _combine_body,
        grid=(_N_LAYERS,),
        in_specs=[
            pl.BlockSpec((1, d, d), lambda i: (i, 0, 0)),
            pl.BlockSpec((1, d, d), lambda i: (i, 0, 0)),
            pl.BlockSpec((1, 1, d), lambda i: (i, 0, 0)),
        ],
        out_specs=[
            pl.BlockSpec((d, d), lambda i: (0, 0)),
            pl.BlockSpec((1, d), lambda i: (0, 0)),
        ],
        out_shape=[
            jax.ShapeDtypeStruct((d, d), jnp.float32),
            jax.ShapeDtypeStruct((1, d), jnp.float32),
        ],
        scratch_shapes=[
            pltpu.VMEM((d, d), jnp.float32),
            pltpu.VMEM((1, d), jnp.float32),
        ],
        compiler_params=pltpu.CompilerParams(
            dimension_semantics=("arbitrary",),
        ),
    )(w_hi, w_lo, bs3)

    return pl.pallas_call(
        _apply_body,
        grid=(n_tok // _BT,),
        in_specs=[
            pl.BlockSpec((_BT, d), lambda t: (t, 0)),
            pl.BlockSpec((d, d), lambda t: (0, 0)),
            pl.BlockSpec((1, d), lambda t: (0, 0)),
        ],
        out_specs=pl.BlockSpec((_BT, d), lambda t: (t, 0)),
        out_shape=jax.ShapeDtypeStruct((n_tok, d), jnp.float32),
        compiler_params=pltpu.CompilerParams(
            dimension_semantics=("parallel",),
        ),
    )(x, wct, bc)


# speedup vs baseline: 5.4592x; 1.3610x over previous
import jax
import jax.numpy as jnp
from jax.experimental import pallas as pl
from jax.experimental.pallas import tpu as pltpu

_N_LAYERS = 20
_D = 768
_BT = 2048
_N_TOK_STEPS = 8


def _dotnn(a, b):
    return jax.lax.dot_general(
        a, b, (((1,), (0,)), ((), ())), preferred_element_type=jnp.float32)


def _dotnt(a, b):
    return jax.lax.dot_general(
        a, b, (((1,), (1,)), ((), ())), preferred_element_type=jnp.float32)


def _body(whi_ref, wlo_ref, b_ref, x_ref, o_ref, m_ref, r_ref):
    i = pl.program_id(0)
    whi = whi_ref[0, :, :]
    wlo = wlo_ref[0, :, :]
    b = b_ref[0, :, :]

    @pl.when(i == 0)
    def _init():
        m_ref[...] = whi.astype(jnp.float32) + wlo.astype(jnp.float32)
        r_ref[...] = b

    @pl.when(jnp.logical_and(i > 0, i < _N_LAYERS))
    def _step():
        m = m_ref[...]
        mhi = m.astype(jnp.bfloat16)
        mlo = (m - mhi.astype(jnp.float32)).astype(jnp.bfloat16)
        m_ref[...] = _dotnn(whi, mhi) + (_dotnn(whi, mlo) + _dotnn(wlo, mhi))
        rb = r_ref[...].astype(jnp.bfloat16)
        r_ref[...] = _dotnt(rb, whi) + _dotnt(rb, wlo) + b

    @pl.when(i >= _N_LAYERS)
    def _apply():
        o_ref[...] = _dotnt(x_ref[...], m_ref[...]) + r_ref[0, :][None, :]


@jax.jit
def kernel(x, Ws, bs):
    n_tok, d = x.shape
    bs3 = bs.reshape(_N_LAYERS, 1, d)
    w_hi = Ws.astype(jnp.bfloat16)
    w_lo = (Ws - w_hi.astype(jnp.float32)).astype(jnp.bfloat16)

    def wmap(i):
        return (jnp.minimum(i, _N_LAYERS - 1), 0, 0)

    def xmap(i):
        return (jnp.maximum(i - _N_LAYERS, 0), 0)

    return pl.pallas_call(
        _body,
        grid=(_N_LAYERS + n_tok // _BT,),
        in_specs=[
            pl.BlockSpec((1, d, d), wmap),
            pl.BlockSpec((1, d, d), wmap),
            pl.BlockSpec((1, 1, d), wmap),
            pl.BlockSpec((_BT, d), xmap),
        ],
        out_specs=pl.BlockSpec((_BT, d), xmap),
        out_shape=jax.ShapeDtypeStruct((n_tok, d), jnp.float32),
        scratch_shapes=[
            pltpu.VMEM((d, d), jnp.float32),
            pltpu.VMEM((1, d), jnp.float32),
        ],
        compiler_params=pltpu.CompilerParams(
            dimension_semantics=("arbitrary",),
        ),
    )(w_hi, w_lo, bs3, x)


# speedup vs baseline: 5.6543x; 1.0357x over previous
import jax
import jax.numpy as jnp
from jax.experimental import pallas as pl
from jax.experimental.pallas import tpu as pltpu

_N_LAYERS = 20
_D = 768
_BT = 2048
_N_TOK_STEPS = 8


def _dotnn(a, b):
    return jax.lax.dot_general(
        a, b, (((1,), (0,)), ((), ())), preferred_element_type=jnp.float32)


def _dotnt(a, b):
    return jax.lax.dot_general(
        a, b, (((1,), (1,)), ((), ())), preferred_element_type=jnp.float32)


def _body(whi_ref, wlo_ref, b_ref, x_ref, o_ref, m_ref, r_ref):
    i = pl.program_id(0)
    whi = whi_ref[0, :, :]
    wlo = wlo_ref[0, :, :]
    b = b_ref[0, :, :]

    @pl.when(i == 0)
    def _init():
        m_ref[...] = whi.astype(jnp.float32) + wlo.astype(jnp.float32)
        r_ref[...] = b

    @pl.when(jnp.logical_and(i > 0, i < _N_LAYERS))
    def _step():
        m = m_ref[...]
        mhi = m.astype(jnp.bfloat16)
        mlo = (m - mhi.astype(jnp.float32)).astype(jnp.bfloat16)
        m_ref[...] = _dotnn(whi, mhi) + (_dotnn(whi, mlo) + _dotnn(wlo, mhi))
        rb = r_ref[...].astype(jnp.bfloat16)
        r_ref[...] = _dotnt(rb, whi) + b

    @pl.when(i >= _N_LAYERS)
    def _apply():
        o_ref[...] = _dotnt(x_ref[...], m_ref[...]) + r_ref[0, :][None, :]


@jax.jit
def kernel(x, Ws, bs):
    n_tok, d = x.shape
    bs3 = bs.reshape(_N_LAYERS, 1, d)
    w_hi = Ws.astype(jnp.bfloat16)
    w_lo = (Ws - w_hi.astype(jnp.float32)).astype(jnp.bfloat16)

    def wmap(i):
        return (jnp.minimum(i, _N_LAYERS - 1), 0, 0)

    def xmap(i):
        return (jnp.maximum(i - _N_LAYERS, 0), 0)

    return pl.pallas_call(
        _body,
        grid=(_N_LAYERS + n_tok // _BT,),
        in_specs=[
            pl.BlockSpec((1, d, d), wmap),
            pl.BlockSpec((1, d, d), wmap),
            pl.BlockSpec((1, 1, d), wmap),
            pl.BlockSpec((_BT, d), xmap),
        ],
        out_specs=pl.BlockSpec((_BT, d), xmap),
        out_shape=jax.ShapeDtypeStruct((n_tok, d), jnp.float32),
        scratch_shapes=[
            pltpu.VMEM((d, d), jnp.float32),
            pltpu.VMEM((1, d), jnp.float32),
        ],
        compiler_params=pltpu.CompilerParams(
            dimension_semantics=("arbitrary",),
        ),
    )(w_hi, w_lo, bs3, x)
